# baseline (speedup 1.0000x reference)
import functools

import jax
import jax.numpy as jnp
from jax import lax
from jax.experimental import pallas as pl
from jax.experimental.pallas import tpu as pltpu
from jax.experimental.pallas import tpu_sc as plsc

_BINS = 30
_BM = 512
_CH = 16
_LANES = 1024
_KS = list(range(1, _BINS))

_NC = 2
_NS = 16
_NW = _NC * _NS
_GW = 128
_GCH = 128


def _sc_gather_body(tab_hbm, idx_hbm, out_hbm, idx_v, rows_v, sem):
    wid = lax.axis_index("s") * _NC + lax.axis_index("c")
    b_per_w = idx_v.shape[0]
    base = wid * b_per_w
    pltpu.sync_copy(idx_hbm.at[pl.ds(base, b_per_w)], idx_v)
    for c in range(b_per_w // _GCH):
        pltpu.async_copy(
            tab_hbm.at[idx_v.at[pl.ds(c * _GCH, _GCH)]], rows_v, sem).wait()
        pltpu.sync_copy(rows_v, out_hbm.at[pl.ds(base + c * _GCH, _GCH)])


def _gather_weights(weight, target):
    B = target.shape[0]
    C = weight.shape[0]
    b_per_w = B // _NW
    wtab = jnp.broadcast_to(weight[:, None], (C, _GW))
    mesh = plsc.VectorSubcoreMesh(core_axis_name="c", subcore_axis_name="s")
    rows = pl.kernel(
        _sc_gather_body,
        out_type=jax.ShapeDtypeStruct((B, _GW), jnp.float32),
        mesh=mesh,
        scratch_types=[
            pltpu.VMEM((b_per_w,), jnp.int32),
            pltpu.VMEM((_GCH, _GW), jnp.float32),
            pltpu.SemaphoreType.DMA,
        ],
    )(wtab, target)
    return rows


def _ghm_block(pred_ref, tgt_ref, ws_ref, out_ref, u_scr, v_scr, acc_ref, *, tot):
    i = pl.program_id(0)
    nblk = pl.num_programs(0)

    @pl.when(i == 0)
    def _init():
        for k in range(_BINS):
            acc_ref[0, k] = 0.0
            acc_ref[1, k] = 0.0
        acc_ref[2, 0] = 0.0
        acc_ref[2, 1] = 0.0

    p = pred_ref[...]
    t = tgt_ref[...]
    w_row = ws_ref[:, 0:1]
    C = p.shape[1]

    col = jax.lax.broadcasted_iota(jnp.int32, p.shape, 1)
    onehot = col == t

    ap = jnp.abs(p)
    e = jnp.exp(-ap)
    r = 1.0 / (1.0 + e)
    s = jnp.where(p >= 0, r, e * r)
    g = jnp.where(onehot, 1.0 - s, s)
    loss = jnp.maximum(p, 0.0) + jnp.log1p(e) - jnp.where(onehot, p, 0.0)

    val = loss * w_row
    u = g * _BINS

    pad = _LANES - C
    u_scr[...] = jnp.concatenate(
        [u, jnp.full((p.shape[0], pad), 1e9, jnp.float32)], axis=1)
    v_scr[...] = jnp.concatenate(
        [val, jnp.zeros((p.shape[0], pad), jnp.float32)], axis=1)

    acc_ref[2, 0] += jnp.sum(w_row)
    acc_ref[2, 1] += jnp.sum(val)

    nchunk = p.shape[0] // _CH
    zero = jnp.zeros((8, 128), jnp.float32)
    init = tuple(zero for _ in range(2 * len(_KS)))

    def body(c, carry):
        accs = list(carry)
        uc = u_scr[pl.ds(c * _CH, _CH), :]
        vc = v_scr[pl.ds(c * _CH, _CH), :]
        for sr in range(_CH // 8):
            rs = slice(sr * 8, (sr + 1) * 8)
            for lg in range(_LANES // 128):
                sl = slice(lg * 128, (lg + 1) * 128)
                u_sl = uc[rs, sl]
                v_sl = vc[rs, sl]
                for j, k in enumerate(_KS):
                    m = u_sl < float(k)
                    accs[2 * j] = accs[2 * j] + jnp.where(m, 1.0, 0.0)
                    accs[2 * j + 1] = accs[2 * j + 1] + jnp.where(m, v_sl, 0.0)
        return tuple(accs)

    res = jax.lax.fori_loop(0, nchunk, body, init)
    for j, k in enumerate(_KS):
        acc_ref[0, k] += jnp.sum(res[2 * j])
        acc_ref[1, k] += jnp.sum(res[2 * j + 1])

    @pl.when(i == nblk - 1)
    def _fin():
        n_elems = jnp.float32(tot)
        total = jnp.float32(0.0)
        n = jnp.float32(0.0)
        for b in range(_BINS):
            c_lo = acc_ref[0, b] if b > 0 else jnp.float32(0.0)
            c_hi = acc_ref[0, b + 1] if b + 1 < _BINS else n_elems
            s_lo = acc_ref[1, b] if b > 0 else jnp.float32(0.0)
            s_hi = acc_ref[1, b + 1] if b + 1 < _BINS else acc_ref[2, 1]
            cnt = c_hi - c_lo
            n += jnp.where(cnt > 0.0, 1.0, 0.0)
            total += (s_hi - s_lo) / jnp.maximum(cnt, 1.0)
        wsum = acc_ref[2, 0] * C
        out_ref[0, 0] = (tot / n) * total / wsum


def kernel(pred, target, weight):
    B, C = pred.shape
    nblk = B // _BM
    t2 = target.reshape(B, 1)
    ws = _gather_weights(weight, target)
    out = pl.pallas_call(
        functools.partial(_ghm_block, tot=float(B * C)),
        grid=(nblk,),
        in_specs=[
            pl.BlockSpec((_BM, C), lambda i: (i, 0)),
            pl.BlockSpec((_BM, 1), lambda i: (i, 0)),
            pl.BlockSpec((_BM, _GW), lambda i: (i, 0)),
        ],
        out_specs=pl.BlockSpec(memory_space=pltpu.SMEM),
        out_shape=jax.ShapeDtypeStruct((1, 1), jnp.float32),
        scratch_shapes=[
            pltpu.VMEM((_BM, _LANES), jnp.float32),
            pltpu.VMEM((_BM, _LANES), jnp.float32),
            pltpu.SMEM((4, _BINS + 2), jnp.float32),
        ],
    )(pred, t2, ws)
    return out[0, 0]


# speedup vs baseline: 1.0076x; 1.0076x over previous
import functools

import jax
import jax.numpy as jnp
from jax import lax
from jax.experimental import pallas as pl
from jax.experimental.pallas import tpu as pltpu
from jax.experimental.pallas import tpu_sc as plsc

_BINS = 30
_BM = 512
_CH = 16
_LANES = 1024
_KS = list(range(1, _BINS))

_NC = 2
_NS = 16
_NW = _NC * _NS
_GW = 128
_GCH = 128


def _sc_gather_body(tab_hbm, idx_hbm, out_hbm, idx_v, rows_v, sem):
    wid = lax.axis_index("s") * _NC + lax.axis_index("c")
    b_per_w = idx_v.shape[0]
    base = wid * b_per_w
    pltpu.sync_copy(idx_hbm.at[pl.ds(base, b_per_w)], idx_v)
    for c in range(b_per_w // _GCH):
        pltpu.async_copy(
            tab_hbm.at[idx_v.at[pl.ds(c * _GCH, _GCH)]], rows_v, sem).wait()
        pltpu.sync_copy(rows_v, out_hbm.at[pl.ds(base + c * _GCH, _GCH)])


def _gather_weights(weight, target):
    B = target.shape[0]
    C = weight.shape[0]
    b_per_w = B // _NW
    wtab = jnp.broadcast_to(weight[:, None], (C, _GW))
    mesh = plsc.VectorSubcoreMesh(core_axis_name="c", subcore_axis_name="s")
    rows = pl.kernel(
        _sc_gather_body,
        out_type=jax.ShapeDtypeStruct((B, _GW), jnp.float32),
        mesh=mesh,
        scratch_types=[
            pltpu.VMEM((b_per_w,), jnp.int32),
            pltpu.VMEM((_GCH, _GW), jnp.float32),
            pltpu.SemaphoreType.DMA,
        ],
    )(wtab, target)
    return rows


def _ghm_block(pred_ref, tgt_ref, ws_ref, out_ref, u_scr, v_scr, accv, acc_ref,
               *, tot):
    i = pl.program_id(0)
    nblk = pl.num_programs(0)

    @pl.when(i == 0)
    def _init():
        acc_ref[2, 0] = 0.0
        acc_ref[2, 1] = 0.0
        accv[...] = jnp.zeros_like(accv)

    p = pred_ref[...]
    t = tgt_ref[...]
    w_row = ws_ref[:, 0:1]
    C = p.shape[1]

    col = jax.lax.broadcasted_iota(jnp.int32, p.shape, 1)
    onehot = col == t

    ap = jnp.abs(p)
    e = jnp.exp(-ap)
    r = 1.0 / (1.0 + e)
    s = jnp.where(p >= 0, r, e * r)
    g = jnp.where(onehot, 1.0 - s, s)
    loss = jnp.maximum(p, 0.0) + jnp.log1p(e) - jnp.where(onehot, p, 0.0)

    val = loss * w_row
    u = g * _BINS

    pad = _LANES - C
    u_scr[...] = jnp.concatenate(
        [u, jnp.full((p.shape[0], pad), 1e9, jnp.float32)], axis=1)
    v_scr[...] = jnp.concatenate(
        [val, jnp.zeros((p.shape[0], pad), jnp.float32)], axis=1)

    acc_ref[2, 0] += jnp.sum(w_row)
    acc_ref[2, 1] += jnp.sum(val)

    nchunk = p.shape[0] // _CH
    zero = jnp.zeros((8, 128), jnp.float32)
    init = tuple(zero for _ in range(2 * len(_KS)))

    def body(c, carry):
        accs = list(carry)
        uc = u_scr[pl.ds(c * _CH, _CH), :]
        vc = v_scr[pl.ds(c * _CH, _CH), :]
        for sr in range(_CH // 8):
            rs = slice(sr * 8, (sr + 1) * 8)
            for lg in range(_LANES // 128):
                sl = slice(lg * 128, (lg + 1) * 128)
                u_sl = uc[rs, sl]
                v_sl = vc[rs, sl]
                for j, k in enumerate(_KS):
                    m = u_sl < float(k)
                    accs[2 * j] = accs[2 * j] + jnp.where(m, 1.0, 0.0)
                    accs[2 * j + 1] = accs[2 * j + 1] + jnp.where(m, v_sl, 0.0)
        return tuple(accs)

    res = jax.lax.fori_loop(0, nchunk, body, init)
    for j in range(2 * len(_KS)):
        accv[j] = accv[j] + res[j]

    @pl.when(i == nblk - 1)
    def _fin():
        n_elems = jnp.float32(tot)
        cum_c = [jnp.float32(0.0)] * (_BINS + 1)
        cum_s = [jnp.float32(0.0)] * (_BINS + 1)
        for j, k in enumerate(_KS):
            cum_c[k] = jnp.sum(accv[2 * j])
            cum_s[k] = jnp.sum(accv[2 * j + 1])
        cum_c[_BINS] = n_elems
        cum_s[_BINS] = acc_ref[2, 1]
        total = jnp.float32(0.0)
        n = jnp.float32(0.0)
        for b in range(_BINS):
            cnt = cum_c[b + 1] - cum_c[b]
            n += jnp.where(cnt > 0.0, 1.0, 0.0)
            total += (cum_s[b + 1] - cum_s[b]) / jnp.maximum(cnt, 1.0)
        wsum = acc_ref[2, 0] * C
        out_ref[0, 0] = (tot / n) * total / wsum


def kernel(pred, target, weight):
    B, C = pred.shape
    nblk = B // _BM
    t2 = target.reshape(B, 1)
    ws = _gather_weights(weight, target)
    out = pl.pallas_call(
        functools.partial(_ghm_block, tot=float(B * C)),
        grid=(nblk,),
        in_specs=[
            pl.BlockSpec((_BM, C), lambda i: (i, 0)),
            pl.BlockSpec((_BM, 1), lambda i: (i, 0)),
            pl.BlockSpec((_BM, _GW), lambda i: (i, 0)),
        ],
        out_specs=pl.BlockSpec(memory_space=pltpu.SMEM),
        out_shape=jax.ShapeDtypeStruct((1, 1), jnp.float32),
        scratch_shapes=[
            pltpu.VMEM((_BM, _LANES), jnp.float32),
            pltpu.VMEM((_BM, _LANES), jnp.float32),
            pltpu.VMEM((2 * len(_KS), 8, 128), jnp.float32),
            pltpu.SMEM((4, _BINS + 2), jnp.float32),
        ],
    )(pred, t2, ws)
    return out[0, 0]


# speedup vs baseline: 1.0206x; 1.0129x over previous
import functools

import jax
import jax.numpy as jnp
from jax import lax
from jax.experimental import pallas as pl
from jax.experimental.pallas import tpu as pltpu
from jax.experimental.pallas import tpu_sc as plsc

_BINS = 30
_BM = 512
_CH = 32
_LANES = 1024
_KS = list(range(1, _BINS))

_NC = 2
_NS = 16
_NW = _NC * _NS
_GW = 128
_GCH = 128


def _sc_gather_body(tab_hbm, idx_hbm, out_hbm, idx_v, rows_v, sem):
    wid = lax.axis_index("s") * _NC + lax.axis_index("c")
    b_per_w = idx_v.shape[0]
    base = wid * b_per_w
    pltpu.sync_copy(idx_hbm.at[pl.ds(base, b_per_w)], idx_v)
    for c in range(b_per_w // _GCH):
        pltpu.async_copy(
            tab_hbm.at[idx_v.at[pl.ds(c * _GCH, _GCH)]], rows_v, sem).wait()
        pltpu.sync_copy(rows_v, out_hbm.at[pl.ds(base + c * _GCH, _GCH)])


def _gather_weights(weight, target):
    B = target.shape[0]
    C = weight.shape[0]
    b_per_w = B // _NW
    wtab = jnp.broadcast_to(weight[:, None], (C, _GW))
    mesh = plsc.VectorSubcoreMesh(core_axis_name="c", subcore_axis_name="s")
    rows = pl.kernel(
        _sc_gather_body,
        out_type=jax.ShapeDtypeStruct((B, _GW), jnp.float32),
        mesh=mesh,
        scratch_types=[
            pltpu.VMEM((b_per_w,), jnp.int32),
            pltpu.VMEM((_GCH, _GW), jnp.float32),
            pltpu.SemaphoreType.DMA,
        ],
    )(wtab, target)
    return rows


def _ghm_block(pred_ref, tgt_ref, ws_ref, out_ref, u_scr, v_scr, accv, acc_ref,
               *, tot):
    i = pl.program_id(0)
    nblk = pl.num_programs(0)

    @pl.when(i == 0)
    def _init():
        acc_ref[2, 0] = 0.0
        acc_ref[2, 1] = 0.0
        accv[...] = jnp.zeros_like(accv)

    p = pred_ref[...]
    t = tgt_ref[...]
    w_row = ws_ref[:, 0:1]
    C = p.shape[1]

    col = jax.lax.broadcasted_iota(jnp.int32, p.shape, 1)
    onehot = col == t

    ap = jnp.abs(p)
    e = jnp.exp(-ap)
    r = 1.0 / (1.0 + e)
    s = jnp.where(p >= 0, r, e * r)
    g = jnp.where(onehot, 1.0 - s, s)
    loss = jnp.maximum(p, 0.0) + jnp.log1p(e) - jnp.where(onehot, p, 0.0)

    val = loss * w_row
    u = g * _BINS

    pad = _LANES - C
    u_scr[...] = jnp.concatenate(
        [u, jnp.full((p.shape[0], pad), 1e9, jnp.float32)], axis=1)
    v_scr[...] = jnp.concatenate(
        [val, jnp.zeros((p.shape[0], pad), jnp.float32)], axis=1)

    acc_ref[2, 0] += jnp.sum(w_row)
    acc_ref[2, 1] += jnp.sum(val)

    nchunk = p.shape[0] // _CH
    zero = jnp.zeros((8, 128), jnp.float32)
    init = tuple(zero for _ in range(2 * len(_KS)))

    def body(c, carry):
        accs = list(carry)
        uc = u_scr[pl.ds(c * _CH, _CH), :]
        vc = v_scr[pl.ds(c * _CH, _CH), :]
        for sr in range(_CH // 8):
            rs = slice(sr * 8, (sr + 1) * 8)
            for lg in range(_LANES // 128):
                sl = slice(lg * 128, (lg + 1) * 128)
                u_sl = uc[rs, sl]
                v_sl = vc[rs, sl]
                for j, k in enumerate(_KS):
                    m = u_sl < float(k)
                    accs[2 * j] = accs[2 * j] + jnp.where(m, 1.0, 0.0)
                    accs[2 * j + 1] = accs[2 * j + 1] + jnp.where(m, v_sl, 0.0)
        return tuple(accs)

    res = jax.lax.fori_loop(0, nchunk, body, init)
    for j in range(2 * len(_KS)):
        accv[j] = accv[j] + res[j]

    @pl.when(i == nblk - 1)
    def _fin():
        n_elems = jnp.float32(tot)
        cum_c = [jnp.float32(0.0)] * (_BINS + 1)
        cum_s = [jnp.float32(0.0)] * (_BINS + 1)
        for j, k in enumerate(_KS):
            cum_c[k] = jnp.sum(accv[2 * j])
            cum_s[k] = jnp.sum(accv[2 * j + 1])
        cum_c[_BINS] = n_elems
        cum_s[_BINS] = acc_ref[2, 1]
        total = jnp.float32(0.0)
        n = jnp.float32(0.0)
        for b in range(_BINS):
            cnt = cum_c[b + 1] - cum_c[b]
            n += jnp.where(cnt > 0.0, 1.0, 0.0)
            total += (cum_s[b + 1] - cum_s[b]) / jnp.maximum(cnt, 1.0)
        wsum = acc_ref[2, 0] * C
        out_ref[0, 0] = (tot / n) * total / wsum


def kernel(pred, target, weight):
    B, C = pred.shape
    nblk = B // _BM
    t2 = target.reshape(B, 1)
    ws = _gather_weights(weight, target)
    out = pl.pallas_call(
        functools.partial(_ghm_block, tot=float(B * C)),
        grid=(nblk,),
        in_specs=[
            pl.BlockSpec((_BM, C), lambda i: (i, 0)),
            pl.BlockSpec((_BM, 1), lambda i: (i, 0)),
            pl.BlockSpec((_BM, _GW), lambda i: (i, 0)),
        ],
        out_specs=pl.BlockSpec(memory_space=pltpu.SMEM),
        out_shape=jax.ShapeDtypeStruct((1, 1), jnp.float32),
        scratch_shapes=[
            pltpu.VMEM((_BM, _LANES), jnp.float32),
            pltpu.VMEM((_BM, _LANES), jnp.float32),
            pltpu.VMEM((2 * len(_KS), 8, 128), jnp.float32),
            pltpu.SMEM((4, _BINS + 2), jnp.float32),
        ],
    )(pred, t2, ws)
    return out[0, 0]


# speedup vs baseline: 1.0247x; 1.0040x over previous
import functools

import jax
import jax.numpy as jnp
from jax import lax
from jax.experimental import pallas as pl
from jax.experimental.pallas import tpu as pltpu
from jax.experimental.pallas import tpu_sc as plsc

_BINS = 30
_BM = 512
_CH = 64
_LANES = 1024
_KS = list(range(1, _BINS))

_NC = 2
_NS = 16
_NW = _NC * _NS
_GW = 128
_GCH = 128


def _sc_gather_body(tab_hbm, idx_hbm, out_hbm, idx_v, rows_v, sem):
    wid = lax.axis_index("s") * _NC + lax.axis_index("c")
    b_per_w = idx_v.shape[0]
    base = wid * b_per_w
    pltpu.sync_copy(idx_hbm.at[pl.ds(base, b_per_w)], idx_v)
    for c in range(b_per_w // _GCH):
        pltpu.async_copy(
            tab_hbm.at[idx_v.at[pl.ds(c * _GCH, _GCH)]], rows_v, sem).wait()
        pltpu.sync_copy(rows_v, out_hbm.at[pl.ds(base + c * _GCH, _GCH)])


def _gather_weights(weight, target):
    B = target.shape[0]
    C = weight.shape[0]
    b_per_w = B // _NW
    wtab = jnp.broadcast_to(weight[:, None], (C, _GW))
    mesh = plsc.VectorSubcoreMesh(core_axis_name="c", subcore_axis_name="s")
    rows = pl.kernel(
        _sc_gather_body,
        out_type=jax.ShapeDtypeStruct((B, _GW), jnp.float32),
        mesh=mesh,
        scratch_types=[
            pltpu.VMEM((b_per_w,), jnp.int32),
            pltpu.VMEM((_GCH, _GW), jnp.float32),
            pltpu.SemaphoreType.DMA,
        ],
    )(wtab, target)
    return rows


def _ghm_block(pred_ref, tgt_ref, ws_ref, out_ref, u_scr, v_scr, accv, acc_ref,
               *, tot):
    i = pl.program_id(0)
    nblk = pl.num_programs(0)

    @pl.when(i == 0)
    def _init():
        acc_ref[2, 0] = 0.0
        acc_ref[2, 1] = 0.0
        accv[...] = jnp.zeros_like(accv)

    p = pred_ref[...]
    t = tgt_ref[...]
    w_row = ws_ref[:, 0:1]
    C = p.shape[1]

    col = jax.lax.broadcasted_iota(jnp.int32, p.shape, 1)
    onehot = col == t

    ap = jnp.abs(p)
    e = jnp.exp(-ap)
    r = 1.0 / (1.0 + e)
    s = jnp.where(p >= 0, r, e * r)
    g = jnp.where(onehot, 1.0 - s, s)
    loss = jnp.maximum(p, 0.0) + jnp.log1p(e) - jnp.where(onehot, p, 0.0)

    val = loss * w_row
    u = g * _BINS

    pad = _LANES - C
    u_scr[...] = jnp.concatenate(
        [u, jnp.full((p.shape[0], pad), 1e9, jnp.float32)], axis=1)
    v_scr[...] = jnp.concatenate(
        [val, jnp.zeros((p.shape[0], pad), jnp.float32)], axis=1)

    acc_ref[2, 0] += jnp.sum(w_row)
    acc_ref[2, 1] += jnp.sum(val)

    nchunk = p.shape[0] // _CH
    zero = jnp.zeros((8, 128), jnp.float32)
    init = tuple(zero for _ in range(2 * len(_KS)))

    def body(c, carry):
        accs = list(carry)
        uc = u_scr[pl.ds(c * _CH, _CH), :]
        vc = v_scr[pl.ds(c * _CH, _CH), :]
        for sr in range(_CH // 8):
            rs = slice(sr * 8, (sr + 1) * 8)
            for lg in range(_LANES // 128):
                sl = slice(lg * 128, (lg + 1) * 128)
                u_sl = uc[rs, sl]
                v_sl = vc[rs, sl]
                for j, k in enumerate(_KS):
                    m = u_sl < float(k)
                    accs[2 * j] = accs[2 * j] + jnp.where(m, 1.0, 0.0)
                    accs[2 * j + 1] = accs[2 * j + 1] + jnp.where(m, v_sl, 0.0)
        return tuple(accs)

    res = jax.lax.fori_loop(0, nchunk, body, init)
    for j in range(2 * len(_KS)):
        accv[j] = accv[j] + res[j]

    @pl.when(i == nblk - 1)
    def _fin():
        n_elems = jnp.float32(tot)
        cum_c = [jnp.float32(0.0)] * (_BINS + 1)
        cum_s = [jnp.float32(0.0)] * (_BINS + 1)
        for j, k in enumerate(_KS):
            cum_c[k] = jnp.sum(accv[2 * j])
            cum_s[k] = jnp.sum(accv[2 * j + 1])
        cum_c[_BINS] = n_elems
        cum_s[_BINS] = acc_ref[2, 1]
        total = jnp.float32(0.0)
        n = jnp.float32(0.0)
        for b in range(_BINS):
            cnt = cum_c[b + 1] - cum_c[b]
            n += jnp.where(cnt > 0.0, 1.0, 0.0)
            total += (cum_s[b + 1] - cum_s[b]) / jnp.maximum(cnt, 1.0)
        wsum = acc_ref[2, 0] * C
        out_ref[0, 0] = (tot / n) * total / wsum


def kernel(pred, target, weight):
    B, C = pred.shape
    nblk = B // _BM
    t2 = target.reshape(B, 1)
    ws = _gather_weights(weight, target)
    out = pl.pallas_call(
        functools.partial(_ghm_block, tot=float(B * C)),
        grid=(nblk,),
        in_specs=[
            pl.BlockSpec((_BM, C), lambda i: (i, 0)),
            pl.BlockSpec((_BM, 1), lambda i: (i, 0)),
            pl.BlockSpec((_BM, _GW), lambda i: (i, 0)),
        ],
        out_specs=pl.BlockSpec(memory_space=pltpu.SMEM),
        out_shape=jax.ShapeDtypeStruct((1, 1), jnp.float32),
        scratch_shapes=[
            pltpu.VMEM((_BM, _LANES), jnp.float32),
            pltpu.VMEM((_BM, _LANES), jnp.float32),
            pltpu.VMEM((2 * len(_KS), 8, 128), jnp.float32),
            pltpu.SMEM((4, _BINS + 2), jnp.float32),
        ],
    )(pred, t2, ws)
    return out[0, 0]
